# baseline (speedup 1.0000x reference)
import functools

import jax
import jax.numpy as jnp
from jax import lax
from jax.experimental import pallas as pl
from jax.experimental.pallas import tpu as pltpu
from jax.experimental.pallas import tpu_sc as plsc

_N = 10000
_E = 320000
_D_IN = 128
_D_HID = 16
_D_OUT = 40

_NC = 2
_NS = 16
_NW = _NC * _NS
_EPW = _E // _NW
_CH = 512
_NFULL = _EPW // _CH
_TAIL = _EPW - _NFULL * _CH
_RPS = _N // _NS
_NBUF = 4

_NV = _N // 8

_SC_PARAMS = pltpu.CompilerParams(use_tc_tiling_on_sc=False)
_MESH = plsc.VectorSubcoreMesh(core_axis_name="c", subcore_axis_name="s")


def _prop_body(g_hbm, ei_hbm, out0_hbm, out1_hbm,
               row_v, col_v,
               rows_a, rows_b, rows_c, rows_d,
               acc_sh,
               gsem_a, gsem_b, gsem_c, gsem_d,
               ssem_a, ssem_b, ssem_c, ssem_d):
    c = lax.axis_index("c")
    s = lax.axis_index("s")
    wid = s * _NC + c
    bufs = (rows_a, rows_b, rows_c, rows_d)
    gsems = (gsem_a, gsem_b, gsem_c, gsem_d)
    ssems = (ssem_a, ssem_b, ssem_c, ssem_d)

    pltpu.sync_copy(g_hbm.at[pl.ds(s * _RPS, _RPS)],
                    acc_sh.at[pl.ds(s * _RPS, _RPS)])

    base = wid * _EPW
    pltpu.sync_copy(ei_hbm.at[0, pl.ds(base, _EPW)], row_v)
    pltpu.sync_copy(ei_hbm.at[1, pl.ds(base, _EPW)], col_v)
    plsc.subcore_barrier()

    def _gather(j, buf, sem, n):
        return pltpu.async_copy(g_hbm.at[row_v.at[pl.ds(j * _CH, n)]],
                                buf, sem)

    def _wait(j, buf, sem, n):
        pltpu.make_async_copy(g_hbm.at[row_v.at[pl.ds(j * _CH, n)]],
                              buf, sem).wait()

    def _scatter(j, buf, sem, n):
        return pltpu.async_copy(buf, acc_sh.at[col_v.at[pl.ds(j * _CH, n)]],
                                sem, add=True)

    def _swait(j, buf, sem, n):
        pltpu.make_async_copy(buf, acc_sh.at[col_v.at[pl.ds(j * _CH, n)]],
                              sem).wait()

    for b in range(2):
        _gather(b, bufs[b], gsems[b], _CH)

    def _step(j, b):
        _wait(j, bufs[b], gsems[b], _CH)
        _scatter(j, bufs[b], ssems[b], _CH)
        k = j + 2
        bk = (b + 2) % _NBUF

        @pl.when((k >= _NBUF) & (k < _NFULL))
        def _(k=k, bk=bk):
            _swait(k - _NBUF, bufs[bk], ssems[bk], _CH)

        @pl.when(k < _NFULL)
        def _(k=k, bk=bk):
            _gather(k, bufs[bk], gsems[bk], _CH)

    def _block(i, carry):
        for b in range(_NBUF):
            _step(i * _NBUF + b, b)
        return carry

    lax.fori_loop(0, _NFULL // _NBUF, _block, 0)
    for j in range(_NFULL - _NFULL % _NBUF, _NFULL):
        _step(j, j % _NBUF)

    for j in range(_NFULL - _NBUF, _NFULL):
        _swait(j, bufs[j % _NBUF], ssems[j % _NBUF], _CH)

    tail = bufs[0].at[pl.ds(0, _TAIL)]
    _gather(_NFULL, tail, gsems[0], _TAIL)
    _wait(_NFULL, tail, gsems[0], _TAIL)
    pltpu.sync_copy(tail, acc_sh.at[col_v.at[pl.ds(_NFULL * _CH, _TAIL)]],
                    add=True)
    plsc.subcore_barrier()

    @pl.when(c == 0)
    def _():
        pltpu.sync_copy(acc_sh.at[pl.ds(s * _RPS, _RPS)],
                        out0_hbm.at[pl.ds(s * _RPS, _RPS)])

    @pl.when(c == 1)
    def _():
        pltpu.sync_copy(acc_sh.at[pl.ds(s * _RPS, _RPS)],
                        out1_hbm.at[pl.ds(s * _RPS, _RPS)])


_prop = functools.partial(
    pl.kernel,
    out_type=[jax.ShapeDtypeStruct((_N, _D_HID), jnp.float32),
              jax.ShapeDtypeStruct((_N, _D_HID), jnp.float32)],
    scratch_types=(
        [pltpu.VMEM((_EPW,), jnp.int32)] * 2
        + [pltpu.VMEM((_CH, _D_HID), jnp.float32)] * _NBUF
        + [pltpu.VMEM_SHARED((_N, _D_HID), jnp.float32)]
        + [pltpu.SemaphoreType.DMA] * (2 * _NBUF)
    ),
    mesh=_MESH,
    compiler_params=_SC_PARAMS,
)(_prop_body)


def _deg_body(ones_hbm, ei_hbm, out_hbm, col_v, ones_v, acc_sh, sem):
    c = lax.axis_index("c")
    s = lax.axis_index("s")
    wid = s * _NC + c

    @pl.when(s == 0)
    def _():
        pltpu.sync_copy(ones_hbm, acc_sh)

    for k in range(_CH // 16):
        ones_v[pl.ds(k * 16, 16)] = jnp.ones((16,), jnp.float32)
    pltpu.sync_copy(ei_hbm.at[1, pl.ds(wid * _EPW, _EPW)], col_v)
    plsc.subcore_barrier()

    def _chunk(j, carry):
        pltpu.sync_copy(ones_v, acc_sh.at[col_v.at[pl.ds(j * _CH, _CH)]],
                        add=True)
        return carry

    lax.fori_loop(0, _NFULL, _chunk, 0)
    pltpu.sync_copy(ones_v.at[pl.ds(0, _TAIL)],
                    acc_sh.at[col_v.at[pl.ds(_NFULL * _CH, _TAIL)]],
                    add=True)
    plsc.subcore_barrier()

    @pl.when(s == 0)
    def _():
        pltpu.sync_copy(acc_sh, out_hbm.at[c])


_deg = functools.partial(
    pl.kernel,
    out_type=jax.ShapeDtypeStruct((_NC, _N), jnp.float32),
    scratch_types=[
        pltpu.VMEM((_EPW,), jnp.int32),
        pltpu.VMEM((_CH,), jnp.float32),
        pltpu.VMEM_SHARED((_N,), jnp.float32),
        pltpu.SemaphoreType.DMA,
    ],
    mesh=_MESH,
    compiler_params=_SC_PARAMS,
)(_deg_body)


def _mm_body(x8_ref, w1b_ref, h_ref):
    h_ref[...] = jnp.dot(x8_ref[...], w1b_ref[...],
                         preferred_element_type=jnp.float32)


def _scale_body(h_ref, dp0_ref, dp1_ref, g_ref, dv_ref):
    dv = lax.rsqrt(dp0_ref[...] + dp1_ref[...] - 1.0)
    g_ref[...] = h_ref[...] * dv
    dv_ref[...] = dv


def _mid_body(p0_ref, p1_ref, g1_ref, dv_ref, b1_ref, g2_ref):
    s = dv_ref[...] * (p0_ref[...] + p1_ref[...] - g1_ref[...])
    g2_ref[...] = dv_ref[...] * jnp.maximum(s + b1_ref[...], 0.0)


def _fin_body(q0_ref, q1_ref, g2_ref, dv_ref, w2b_ref, b2_ref, out_ref):
    s = dv_ref[...] * (q0_ref[...] + q1_ref[...] - g2_ref[...])
    out_ref[...] = (
        jnp.dot(s, w2b_ref[...], preferred_element_type=jnp.float32)
        + b2_ref[...]
    )


_mm = pl.pallas_call(
    _mm_body,
    out_shape=jax.ShapeDtypeStruct((_NV, 128), jnp.float32),
)

_scale = pl.pallas_call(
    _scale_body,
    out_shape=[jax.ShapeDtypeStruct((_NV, 128), jnp.float32),
               jax.ShapeDtypeStruct((_NV, 128), jnp.float32)],
)

_mid = pl.pallas_call(
    _mid_body,
    out_shape=jax.ShapeDtypeStruct((_NV, 128), jnp.float32),
)

_fin = pl.pallas_call(
    _fin_body,
    out_shape=jax.ShapeDtypeStruct((_NV, 8 * _D_OUT), jnp.float32),
)


def _bdiag(w):
    return jax.scipy.linalg.block_diag(*([w] * 8))


def _view(t):
    return t.reshape(_NV, 128)


def kernel(x, edge_index, W1, b1, W2, b2):
    ei = edge_index.astype(jnp.int32)
    ones_n = jnp.ones((_N,), dtype=jnp.float32)

    h = _mm(x.reshape(_NV, 8 * _D_IN), _bdiag(W1))
    dp = _deg(ones_n, ei)
    def _rep(v):
        return jnp.broadcast_to(
            v.reshape(_NV, 8)[:, :, None], (_NV, 8, _D_HID)
        ).reshape(_NV, 128)

    g1, dv = _scale(h, _rep(dp[0]), _rep(dp[1]))

    p0, p1 = _prop(g1.reshape(_N, _D_HID), ei)
    g2 = _mid(_view(p0), _view(p1), g1, dv, jnp.tile(b1, 8).reshape(1, 128))

    q0, q1 = _prop(g2.reshape(_N, _D_HID), ei)
    out8 = _fin(_view(q0), _view(q1), g2, dv,
                _bdiag(W2), jnp.tile(b2, 8).reshape(1, 8 * _D_OUT))
    return out8.reshape(_N, _D_OUT)


# speedup vs baseline: 1.0094x; 1.0094x over previous
import functools

import jax
import jax.numpy as jnp
from jax import lax
from jax.experimental import pallas as pl
from jax.experimental.pallas import tpu as pltpu
from jax.experimental.pallas import tpu_sc as plsc

_N = 10000
_E = 320000
_D_IN = 128
_D_HID = 16
_D_OUT = 40

_NC = 2
_NS = 16
_NW = _NC * _NS
_EPW = _E // _NW
_CH = 2000
_NFULL = _EPW // _CH
_RPS = _N // _NS
_NBUF = 2

_NV = _N // 8

_SC_PARAMS = pltpu.CompilerParams(use_tc_tiling_on_sc=False)
_MESH = plsc.VectorSubcoreMesh(core_axis_name="c", subcore_axis_name="s")


def _prop_body(g_hbm, ei_hbm, out0_hbm, out1_hbm,
               row_v, col_v, rows_a, rows_b, acc_sh, gsem_a, gsem_b):
    c = lax.axis_index("c")
    s = lax.axis_index("s")
    wid = s * _NC + c
    bufs = (rows_a, rows_b)
    gsems = (gsem_a, gsem_b)

    pltpu.sync_copy(g_hbm.at[pl.ds(s * _RPS, _RPS)],
                    acc_sh.at[pl.ds(s * _RPS, _RPS)])

    base = wid * _EPW
    pltpu.sync_copy(ei_hbm.at[0, pl.ds(base, _EPW)], row_v)
    pltpu.sync_copy(ei_hbm.at[1, pl.ds(base, _EPW)], col_v)
    plsc.subcore_barrier()

    def _gather(j, buf, sem, n):
        return pltpu.async_copy(g_hbm.at[row_v.at[pl.ds(j * _CH, n)]],
                                buf, sem)

    def _wait(j, buf, sem, n):
        pltpu.make_async_copy(g_hbm.at[row_v.at[pl.ds(j * _CH, n)]],
                              buf, sem).wait()

    def _scatter(j, buf, n):
        pltpu.sync_copy(buf, acc_sh.at[col_v.at[pl.ds(j * _CH, n)]],
                        add=True)

    for b in range(_NBUF):
        _gather(b, bufs[b], gsems[b], _CH)
    for j in range(_NFULL):
        b = j % _NBUF
        _wait(j, bufs[b], gsems[b], _CH)
        _scatter(j, bufs[b], _CH)
        if j + _NBUF < _NFULL:
            _gather(j + _NBUF, bufs[b], gsems[b], _CH)
    plsc.subcore_barrier()

    @pl.when(c == 0)
    def _():
        pltpu.sync_copy(acc_sh.at[pl.ds(s * _RPS, _RPS)],
                        out0_hbm.at[pl.ds(s * _RPS, _RPS)])

    @pl.when(c == 1)
    def _():
        pltpu.sync_copy(acc_sh.at[pl.ds(s * _RPS, _RPS)],
                        out1_hbm.at[pl.ds(s * _RPS, _RPS)])


_prop = functools.partial(
    pl.kernel,
    out_type=[jax.ShapeDtypeStruct((_N, _D_HID), jnp.float32),
              jax.ShapeDtypeStruct((_N, _D_HID), jnp.float32)],
    scratch_types=(
        [pltpu.VMEM((_EPW,), jnp.int32)] * 2
        + [pltpu.VMEM((_CH, _D_HID), jnp.float32)] * _NBUF
        + [pltpu.VMEM_SHARED((_N, _D_HID), jnp.float32)]
        + [pltpu.SemaphoreType.DMA] * _NBUF
    ),
    mesh=_MESH,
    compiler_params=_SC_PARAMS,
)(_prop_body)


def _deg_body(ones_hbm, ei_hbm, out_hbm, col_v, ones_v, acc_sh, sem):
    c = lax.axis_index("c")
    s = lax.axis_index("s")
    wid = s * _NC + c

    @pl.when(s == 0)
    def _():
        pltpu.sync_copy(ones_hbm, acc_sh)

    def _fill(k, carry):
        ones_v[pl.ds(k * 16, 16)] = jnp.ones((16,), jnp.float32)
        return carry

    lax.fori_loop(0, _CH // 16, _fill, 0)
    pltpu.sync_copy(ei_hbm.at[1, pl.ds(wid * _EPW, _EPW)], col_v)
    plsc.subcore_barrier()

    for j in range(_NFULL):
        pltpu.sync_copy(ones_v, acc_sh.at[col_v.at[pl.ds(j * _CH, _CH)]],
                        add=True)
    plsc.subcore_barrier()

    @pl.when(s == 0)
    def _():
        pltpu.sync_copy(acc_sh, out_hbm.at[c])


_deg = functools.partial(
    pl.kernel,
    out_type=jax.ShapeDtypeStruct((_NC, _N), jnp.float32),
    scratch_types=[
        pltpu.VMEM((_EPW,), jnp.int32),
        pltpu.VMEM((_CH,), jnp.float32),
        pltpu.VMEM_SHARED((_N,), jnp.float32),
        pltpu.SemaphoreType.DMA,
    ],
    mesh=_MESH,
    compiler_params=_SC_PARAMS,
)(_deg_body)


def _mm_body(x8_ref, w1b_ref, h_ref):
    h_ref[...] = jnp.dot(x8_ref[...], w1b_ref[...],
                         preferred_element_type=jnp.float32)


def _scale_body(h_ref, dp0_ref, dp1_ref, g_ref, dv_ref):
    dv = lax.rsqrt(dp0_ref[...] + dp1_ref[...] - 1.0)
    g_ref[...] = h_ref[...] * dv
    dv_ref[...] = dv


def _mid_body(p0_ref, p1_ref, g1_ref, dv_ref, b1_ref, g2_ref):
    s = dv_ref[...] * (p0_ref[...] + p1_ref[...] - g1_ref[...])
    g2_ref[...] = dv_ref[...] * jnp.maximum(s + b1_ref[...], 0.0)


def _fin_body(q0_ref, q1_ref, g2_ref, dv_ref, w2b_ref, b2_ref, out_ref):
    s = dv_ref[...] * (q0_ref[...] + q1_ref[...] - g2_ref[...])
    out_ref[...] = (
        jnp.dot(s, w2b_ref[...], preferred_element_type=jnp.float32)
        + b2_ref[...]
    )


_mm = pl.pallas_call(
    _mm_body,
    out_shape=jax.ShapeDtypeStruct((_NV, 128), jnp.float32),
)

_scale = pl.pallas_call(
    _scale_body,
    out_shape=[jax.ShapeDtypeStruct((_NV, 128), jnp.float32),
               jax.ShapeDtypeStruct((_NV, 128), jnp.float32)],
)

_mid = pl.pallas_call(
    _mid_body,
    out_shape=jax.ShapeDtypeStruct((_NV, 128), jnp.float32),
)

_fin = pl.pallas_call(
    _fin_body,
    out_shape=jax.ShapeDtypeStruct((_NV, 8 * _D_OUT), jnp.float32),
)


def _bdiag(w):
    return jax.scipy.linalg.block_diag(*([w] * 8))


def _view(t):
    return t.reshape(_NV, 128)


def kernel(x, edge_index, W1, b1, W2, b2):
    ei = edge_index.astype(jnp.int32)
    ones_n = jnp.ones((_N,), dtype=jnp.float32)

    h = _mm(x.reshape(_NV, 8 * _D_IN), _bdiag(W1))
    dp = _deg(ones_n, ei)
    def _rep(v):
        return jnp.broadcast_to(
            v.reshape(_NV, 8)[:, :, None], (_NV, 8, _D_HID)
        ).reshape(_NV, 128)

    g1, dv = _scale(h, _rep(dp[0]), _rep(dp[1]))

    p0, p1 = _prop(g1.reshape(_N, _D_HID), ei)
    g2 = _mid(_view(p0), _view(p1), g1, dv, jnp.tile(b1, 8).reshape(1, 128))

    q0, q1 = _prop(g2.reshape(_N, _D_HID), ei)
    out8 = _fin(_view(q0), _view(q1), g2, dv,
                _bdiag(W2), jnp.tile(b2, 8).reshape(1, 8 * _D_OUT))
    return out8.reshape(_N, _D_OUT)


# speedup vs baseline: 1.0303x; 1.0207x over previous
import functools

import jax
import jax.numpy as jnp
from jax import lax
from jax.experimental import pallas as pl
from jax.experimental.pallas import tpu as pltpu
from jax.experimental.pallas import tpu_sc as plsc

_N = 10000
_E = 320000
_D_IN = 128
_D_HID = 16
_D_OUT = 40

_NC = 2
_NS = 16
_NW = _NC * _NS
_EPW = _E // _NW
_CH = 1000
_NFULL = _EPW // _CH
_RPS = _N // _NS
_NBUF = 4

_NV = _N // 8

_SC_PARAMS = pltpu.CompilerParams(use_tc_tiling_on_sc=False)
_MESH = plsc.VectorSubcoreMesh(core_axis_name="c", subcore_axis_name="s")


def _prop_body(g_hbm, ei_hbm, out0_hbm, out1_hbm,
               row_v, col_v, rows_a, rows_b, rows_c, rows_d, acc_sh,
               gsem_a, gsem_b, gsem_c, gsem_d):
    c = lax.axis_index("c")
    s = lax.axis_index("s")
    wid = s * _NC + c
    bufs = (rows_a, rows_b, rows_c, rows_d)
    gsems = (gsem_a, gsem_b, gsem_c, gsem_d)

    pltpu.sync_copy(g_hbm.at[pl.ds(s * _RPS, _RPS)],
                    acc_sh.at[pl.ds(s * _RPS, _RPS)])

    base = wid * _EPW
    pltpu.sync_copy(ei_hbm.at[0, pl.ds(base, _EPW)], row_v)
    pltpu.sync_copy(ei_hbm.at[1, pl.ds(base, _EPW)], col_v)
    plsc.subcore_barrier()

    def _gather(j, buf, sem, n):
        return pltpu.async_copy(g_hbm.at[row_v.at[pl.ds(j * _CH, n)]],
                                buf, sem)

    def _wait(j, buf, sem, n):
        pltpu.make_async_copy(g_hbm.at[row_v.at[pl.ds(j * _CH, n)]],
                              buf, sem).wait()

    def _scatter(j, buf, n):
        pltpu.sync_copy(buf, acc_sh.at[col_v.at[pl.ds(j * _CH, n)]],
                        add=True)

    for b in range(_NBUF):
        _gather(b, bufs[b], gsems[b], _CH)
    for j in range(_NFULL):
        b = j % _NBUF
        _wait(j, bufs[b], gsems[b], _CH)
        _scatter(j, bufs[b], _CH)
        if j + _NBUF < _NFULL:
            _gather(j + _NBUF, bufs[b], gsems[b], _CH)
    plsc.subcore_barrier()

    @pl.when(c == 0)
    def _():
        pltpu.sync_copy(acc_sh.at[pl.ds(s * _RPS, _RPS)],
                        out0_hbm.at[pl.ds(s * _RPS, _RPS)])

    @pl.when(c == 1)
    def _():
        pltpu.sync_copy(acc_sh.at[pl.ds(s * _RPS, _RPS)],
                        out1_hbm.at[pl.ds(s * _RPS, _RPS)])


_prop = functools.partial(
    pl.kernel,
    out_type=[jax.ShapeDtypeStruct((_N, _D_HID), jnp.float32),
              jax.ShapeDtypeStruct((_N, _D_HID), jnp.float32)],
    scratch_types=(
        [pltpu.VMEM((_EPW,), jnp.int32)] * 2
        + [pltpu.VMEM((_CH, _D_HID), jnp.float32)] * _NBUF
        + [pltpu.VMEM_SHARED((_N, _D_HID), jnp.float32)]
        + [pltpu.SemaphoreType.DMA] * _NBUF
    ),
    mesh=_MESH,
    compiler_params=_SC_PARAMS,
)(_prop_body)


def _deg_body(ones_hbm, ei_hbm, out_hbm, col_v, ones_v, acc_sh, sem):
    c = lax.axis_index("c")
    s = lax.axis_index("s")
    wid = s * _NC + c

    @pl.when(s == 0)
    def _():
        pltpu.sync_copy(ones_hbm, acc_sh)

    def _fill(k, carry):
        ones_v[pl.ds(k * 16, 16)] = jnp.ones((16,), jnp.float32)
        return carry

    lax.fori_loop(0, (_CH + 15) // 16, _fill, 0)
    pltpu.sync_copy(ei_hbm.at[1, pl.ds(wid * _EPW, _EPW)], col_v)
    plsc.subcore_barrier()

    for j in range(_NFULL):
        pltpu.sync_copy(ones_v.at[pl.ds(0, _CH)],
                        acc_sh.at[col_v.at[pl.ds(j * _CH, _CH)]],
                        add=True)
    plsc.subcore_barrier()

    @pl.when(s == 0)
    def _():
        pltpu.sync_copy(acc_sh, out_hbm.at[c])


_deg = functools.partial(
    pl.kernel,
    out_type=jax.ShapeDtypeStruct((_NC, _N), jnp.float32),
    scratch_types=[
        pltpu.VMEM((_EPW,), jnp.int32),
        pltpu.VMEM((((_CH + 15) // 16) * 16,), jnp.float32),
        pltpu.VMEM_SHARED((_N,), jnp.float32),
        pltpu.SemaphoreType.DMA,
    ],
    mesh=_MESH,
    compiler_params=_SC_PARAMS,
)(_deg_body)


def _mm_body(x8_ref, w1b_ref, h_ref):
    h_ref[...] = jnp.dot(x8_ref[...], w1b_ref[...],
                         preferred_element_type=jnp.float32)


def _scale_body(h_ref, dp0_ref, dp1_ref, g_ref, dv_ref):
    dv = lax.rsqrt(dp0_ref[...] + dp1_ref[...] - 1.0)
    g_ref[...] = h_ref[...] * dv
    dv_ref[...] = dv


def _mid_body(p0_ref, p1_ref, g1_ref, dv_ref, b1_ref, g2_ref):
    s = dv_ref[...] * (p0_ref[...] + p1_ref[...] - g1_ref[...])
    g2_ref[...] = dv_ref[...] * jnp.maximum(s + b1_ref[...], 0.0)


def _fin_body(q0_ref, q1_ref, g2_ref, dv_ref, w2b_ref, b2_ref, out_ref):
    s = dv_ref[...] * (q0_ref[...] + q1_ref[...] - g2_ref[...])
    out_ref[...] = (
        jnp.dot(s, w2b_ref[...], preferred_element_type=jnp.float32)
        + b2_ref[...]
    )


_mm = pl.pallas_call(
    _mm_body,
    out_shape=jax.ShapeDtypeStruct((_NV, 128), jnp.float32),
)

_scale = pl.pallas_call(
    _scale_body,
    out_shape=[jax.ShapeDtypeStruct((_NV, 128), jnp.float32),
               jax.ShapeDtypeStruct((_NV, 128), jnp.float32)],
)

_mid = pl.pallas_call(
    _mid_body,
    out_shape=jax.ShapeDtypeStruct((_NV, 128), jnp.float32),
)

_fin = pl.pallas_call(
    _fin_body,
    out_shape=jax.ShapeDtypeStruct((_NV, 8 * _D_OUT), jnp.float32),
)


def _bdiag(w):
    return jax.scipy.linalg.block_diag(*([w] * 8))


def _view(t):
    return t.reshape(_NV, 128)


def kernel(x, edge_index, W1, b1, W2, b2):
    ei = edge_index.astype(jnp.int32)
    ones_n = jnp.ones((_N,), dtype=jnp.float32)

    h = _mm(x.reshape(_NV, 8 * _D_IN), _bdiag(W1))
    dp = _deg(ones_n, ei)
    def _rep(v):
        return jnp.broadcast_to(
            v.reshape(_NV, 8)[:, :, None], (_NV, 8, _D_HID)
        ).reshape(_NV, 128)

    g1, dv = _scale(h, _rep(dp[0]), _rep(dp[1]))

    p0, p1 = _prop(g1.reshape(_N, _D_HID), ei)
    g2 = _mid(_view(p0), _view(p1), g1, dv, jnp.tile(b1, 8).reshape(1, 128))

    q0, q1 = _prop(g2.reshape(_N, _D_HID), ei)
    out8 = _fin(_view(q0), _view(q1), g2, dv,
                _bdiag(W2), jnp.tile(b2, 8).reshape(1, 8 * _D_OUT))
    return out8.reshape(_N, _D_OUT)
